# baseline (speedup 1.0000x reference)
import jax
import jax.numpy as jnp
from jax.experimental import pallas as pl
from jax.experimental.pallas import tpu as pltpu
from jax.experimental.pallas import tpu_sc as plsc

_B, _C, _H, _W = 128, 3, 256, 256
_T = 50
_TPAD = 64
_R = 4

_scalar_mesh = plsc.ScalarSubcoreMesh(axis_name="core", num_cores=2)


def _sc_gather_body(t_hbm, sac_hbm, som_hbm, out_hbm, t_s, tab_s, o_s, sem):
    core = jax.lax.axis_index("core")

    pltpu.async_copy(t_hbm, t_s, sem).wait()

    @pl.when(core == 0)
    def _():
        pltpu.async_copy(sac_hbm, tab_s, sem).wait()

    @pl.when(core == 1)
    def _():
        pltpu.async_copy(som_hbm, tab_s, sem).wait()

    @pl.loop(0, _B)
    def _(i):
        o_s[i] = tab_s[t_s[i]]

    pltpu.async_copy(o_s, out_hbm.at[core], sem).wait()


def _sc_gather(t, sac, som):
    sac_p = jnp.pad(sac, (0, _TPAD - _T))
    som_p = jnp.pad(som, (0, _TPAD - _T))
    f = pl.kernel(
        _sc_gather_body,
        out_type=jax.ShapeDtypeStruct((2, _B), jnp.float32),
        mesh=_scalar_mesh,
        scratch_types=[
            pltpu.SMEM((_B,), jnp.int32),
            pltpu.SMEM((_TPAD,), jnp.float32),
            pltpu.SMEM((_B,), jnp.float32),
            pltpu.SemaphoreType.DMA,
        ],
    )
    return f(t, sac_p, som_p)


def _tc_fma_body(coef_ref, x_ref, n_ref, o_ref):
    i = pl.program_id(0)
    for r in range(_R):
        a = coef_ref[0, i * _R + r]
        b = coef_ref[1, i * _R + r]
        o_ref[r] = a * x_ref[r] + b * n_ref[r]


def kernel(x_start, t, noise, sqrt_alphas_cumprod, sqrt_one_minus_alphas_cumprod):
    coef = _sc_gather(t, sqrt_alphas_cumprod, sqrt_one_minus_alphas_cumprod)

    grid_spec = pltpu.PrefetchScalarGridSpec(
        num_scalar_prefetch=1,
        grid=(_B // _R,),
        in_specs=[
            pl.BlockSpec((_R, _C, _H, _W), lambda i, *_: (i, 0, 0, 0)),
            pl.BlockSpec((_R, _C, _H, _W), lambda i, *_: (i, 0, 0, 0)),
        ],
        out_specs=pl.BlockSpec((_R, _C, _H, _W), lambda i, *_: (i, 0, 0, 0)),
    )

    return pl.pallas_call(
        _tc_fma_body,
        grid_spec=grid_spec,
        out_shape=jax.ShapeDtypeStruct((_B, _C, _H, _W), jnp.float32),
    )(coef, x_start, noise)
